# baseline (speedup 1.0000x reference)
import functools

import jax
import jax.numpy as jnp
from jax import lax
from jax.experimental import pallas as pl
from jax.experimental.pallas import tpu as pltpu
from jax.experimental.pallas import tpu_sc as plsc

_D = 64
_B = 16384
_NC, _NS, _L = 2, 16, 16
_NW = _NC * _NS
_BPW = _B // _NW
_HALF = _BPW // 2

_mesh = plsc.VectorSubcoreMesh(core_axis_name="c", subcore_axis_name="s")


@functools.partial(
    pl.kernel,
    mesh=_mesh,
    out_type=jax.ShapeDtypeStruct((_B,), jnp.float32),
    compiler_params=pltpu.CompilerParams(
        needs_layout_passes=False, use_tc_tiling_on_sc=True),
    scratch_types=[
        pltpu.VMEM((_BPW,), jnp.int32),
        pltpu.VMEM((_BPW,), jnp.int32),
        pltpu.VMEM((_BPW // 2, 2 * _D), jnp.float32),
        pltpu.VMEM((_BPW // 2, 2 * _D), jnp.float32),
        pltpu.VMEM((_BPW,), jnp.float32),
        pltpu.SemaphoreType.DMA,
    ],
)
def _w2v(center_h, context_h, in_h, oute_h, o_h, cidx, xidx, vbuf, ubuf,
         obuf, sem):
    wid = lax.axis_index("s") * _NC + lax.axis_index("c")
    base = wid * _BPW

    pltpu.sync_copy(center_h.at[pl.ds(base, _BPW)], cidx)
    pltpu.sync_copy(context_h.at[pl.ds(base, _BPW)], xidx)

    def issue_async(g, carry):
        cvec = cidx[pl.ds(g * _L, _L)]
        xvec = xidx[pl.ds(g * _L, _L)]
        for k in range(_L):
            p = g * (_L // 2) + k // 2
            off = (k % 2) * _D
            pltpu.async_copy(in_h.at[cvec[k]],
                             vbuf.at[p, pl.ds(off, _D)], sem)
            pltpu.async_copy(oute_h.at[xvec[k]],
                             ubuf.at[p, pl.ds(off, _D)], sem)
        return carry

    lax.fori_loop(0, _HALF // _L, issue_async, 0)

    def issue_sync(g, carry):
        cvec = cidx[pl.ds(g * _L, _L)]
        xvec = xidx[pl.ds(g * _L, _L)]
        for k in range(_L):
            p = g * (_L // 2) + k // 2
            off = (k % 2) * _D
            pltpu.sync_copy(in_h.at[cvec[k]], vbuf.at[p, pl.ds(off, _D)])
            pltpu.sync_copy(oute_h.at[xvec[k]], ubuf.at[p, pl.ds(off, _D)])
        return carry

    lax.fori_loop(_HALF // _L, _BPW // _L, issue_sync, 0)

    pltpu.make_async_copy(in_h.at[pl.ds(0, _HALF // 2)],
                          vbuf.at[pl.ds(0, _HALF // 2)], sem).wait()
    pltpu.make_async_copy(in_h.at[pl.ds(0, _HALF // 2)],
                          ubuf.at[pl.ds(0, _HALF // 2)], sem).wait()

    iota = lax.iota(jnp.int32, _L)
    last = jnp.full((_L,), _L - 1, jnp.int32)

    def group_body(g, carry):
        outv = jnp.zeros((_L,), jnp.float32)
        for k in range(_L):
            p = g * (_L // 2) + k // 2
            off = (k % 2) * _D
            acc = vbuf[p, pl.ds(off, _L)] * ubuf[p, pl.ds(off, _L)]
            for c in range(1, _D // _L):
                acc = acc + (vbuf[p, pl.ds(off + c * _L, _L)]
                             * ubuf[p, pl.ds(off + c * _L, _L)])
            tot = jnp.cumsum(acc)
            bcast = tot.at[last].get(mode="promise_in_bounds")
            outv = jnp.where(iota == k, bcast, outv)
        obuf[pl.ds(g * _L, _L)] = outv
        return carry

    lax.fori_loop(0, _BPW // _L, group_body, 0)

    pltpu.sync_copy(obuf, o_h.at[pl.ds(base, _BPW)])


def kernel(center, context, in_embed, out_embed):
    return _w2v(center.astype(jnp.int32), context.astype(jnp.int32),
                in_embed, out_embed)


# speedup vs baseline: 1.3041x; 1.3041x over previous
import functools

import jax
import jax.numpy as jnp
from jax import lax
from jax.experimental import pallas as pl
from jax.experimental.pallas import tpu as pltpu
from jax.experimental.pallas import tpu_sc as plsc

_D = 64
_B = 16384
_NC, _NS, _L = 2, 16, 16
_NW = _NC * _NS
_BPW = _B // _NW

_mesh = plsc.VectorSubcoreMesh(core_axis_name="c", subcore_axis_name="s")


@functools.partial(
    pl.kernel,
    mesh=_mesh,
    out_type=jax.ShapeDtypeStruct((_B,), jnp.float32),
    compiler_params=pltpu.CompilerParams(
        needs_layout_passes=False, use_tc_tiling_on_sc=True),
    scratch_types=[
        pltpu.VMEM((_BPW,), jnp.int32),
        pltpu.VMEM((_BPW,), jnp.int32),
        pltpu.VMEM((_BPW // 2, 2 * _D), jnp.float32),
        pltpu.VMEM((_BPW // 2, 2 * _D), jnp.float32),
        pltpu.VMEM((_BPW,), jnp.float32),
        pltpu.SemaphoreType.DMA,
        pltpu.SemaphoreType.DMA,
    ],
)
def _w2v(center_h, context_h, in_h, oute_h, o_h, cidx, xidx, vbuf, ubuf,
         obuf, sem_v, sem_u):
    wid = lax.axis_index("s") * _NC + lax.axis_index("c")
    base = wid * _BPW

    pltpu.sync_copy(center_h.at[pl.ds(base, _BPW)], cidx)
    pltpu.sync_copy(context_h.at[pl.ds(base, _BPW)], xidx)

    def issue_body(g, carry):
        cvec = cidx[pl.ds(g * _L, _L)]
        xvec = xidx[pl.ds(g * _L, _L)]
        for k in range(_L):
            p = g * (_L // 2) + k // 2
            off = (k % 2) * _D
            pltpu.async_copy(in_h.at[cvec[k]],
                             vbuf.at[p, pl.ds(off, _D)], sem_v)
            pltpu.async_copy(oute_h.at[xvec[k]],
                             ubuf.at[p, pl.ds(off, _D)], sem_u)
        return carry

    lax.fori_loop(0, _BPW // _L, issue_body, 0)

    pltpu.make_async_copy(in_h.at[pl.ds(0, _BPW // 2)], vbuf, sem_v).wait()
    pltpu.make_async_copy(oute_h.at[pl.ds(0, _BPW // 2)], ubuf, sem_u).wait()

    iota = lax.iota(jnp.int32, _L)
    last = jnp.full((_L,), _L - 1, jnp.int32)

    def group_body(g, carry):
        outv = jnp.zeros((_L,), jnp.float32)
        for k in range(_L):
            p = g * (_L // 2) + k // 2
            off = (k % 2) * _D
            acc = vbuf[p, pl.ds(off, _L)] * ubuf[p, pl.ds(off, _L)]
            for c in range(1, _D // _L):
                acc = acc + (vbuf[p, pl.ds(off + c * _L, _L)]
                             * ubuf[p, pl.ds(off + c * _L, _L)])
            tot = jnp.cumsum(acc)
            bcast = tot.at[last].get(mode="promise_in_bounds")
            outv = jnp.where(iota == k, bcast, outv)
        obuf[pl.ds(g * _L, _L)] = outv
        return carry

    lax.fori_loop(0, _BPW // _L, group_body, 0)

    pltpu.sync_copy(obuf, o_h.at[pl.ds(base, _BPW)])


def kernel(center, context, in_embed, out_embed):
    return _w2v(center.astype(jnp.int32), context.astype(jnp.int32),
                in_embed, out_embed)
